# baseline (device time: 862677 ns/iter reference)
import jax
import jax.numpy as jnp
from jax import lax
from jax.experimental import pallas as pl
from jax.experimental.pallas import tpu as pltpu

TOK = 2048
DM = 4096
VS = 8192
BR = 128
BC = 1024
R = TOK // BR
C = VS // BC
CH = 2048
NS = 3


def kernel(x, W):
    def body(
        x_ref,
        w_ref,
        out_ref,
        logits_ref,
        recv_ref,
        ov_ref,
        s_ref,
        send_sems,
        recv_sems,
        copy_sem,
    ):
        r = pl.program_id(0)
        c = pl.program_id(1)
        my_x = lax.axis_index("x")
        my_y = lax.axis_index("y")
        my_z = lax.axis_index("z")
        nbr = (1 - my_x, my_y, my_z)
        sslot = lax.rem(r, 2)

        def send_desc(s, rr):
            return pltpu.make_async_remote_copy(
                src_ref=logits_ref.at[s],
                dst_ref=recv_ref.at[lax.rem(rr, NS)],
                send_sem=send_sems.at[s],
                recv_sem=recv_sems.at[lax.rem(rr, NS)],
                device_id=nbr,
                device_id_type=pl.DeviceIdType.MESH,
            )

        @pl.when((r == 0) & (c == 0))
        def _():
            barrier = pltpu.get_barrier_semaphore()
            pl.semaphore_signal(
                barrier, inc=1, device_id=nbr,
                device_id_type=pl.DeviceIdType.MESH,
            )
            pl.semaphore_wait(barrier, 1)

        @pl.when((c == 0) & (r >= 2) & (r < R))
        def _():
            send_desc(sslot, r - 2).wait_send()

        @pl.when((c == 0) & (r == R))
        def _():
            send_desc(lax.rem(R - 2, 2), R - 2).wait_send()
            send_desc(lax.rem(R - 1, 2), R - 1).wait_send()

        @pl.when(r < R)
        def _():
            et = jnp.exp(
                jnp.dot(
                    x_ref[...], w_ref[...],
                    preferred_element_type=jnp.float32,
                )
            )
            logits_ref[sslot, :, pl.ds(c * BC, BC)] = et.astype(jnp.bfloat16)
            rs = jnp.sum(et, axis=1, keepdims=True)
            s_ref[sslot] = jnp.where(c == 0, rs, s_ref[sslot] + rs)

        @pl.when((c == C - 1) & (r < R))
        def _():
            send_desc(sslot, r).start()

        @pl.when((c == C - 1) & (r >= 1))
        def _():
            rp = r - 1
            pslot = lax.rem(rp, 2)
            rslot = lax.rem(rp, NS)
            send_desc(pslot, rp).wait_recv()

            off_loc = my_x * VS
            off_oth = (1 - my_x) * VS

            s = s_ref[pslot]
            for k in range(VS // CH):
                t = recv_ref[rslot, :, pl.ds(k * CH, CH)]
                s = s + jnp.sum(t.astype(jnp.float32), axis=1, keepdims=True)

            inv = 1.0 / s
            for k in range(VS // CH):
                t = logits_ref[pslot, :, pl.ds(k * CH, CH)]
                ov_ref[:, pl.ds(off_loc + k * CH, CH)] = (
                    t.astype(jnp.float32) * inv
                )
                t = recv_ref[rslot, :, pl.ds(k * CH, CH)]
                ov_ref[:, pl.ds(off_oth + k * CH, CH)] = (
                    t.astype(jnp.float32) * inv
                )

            st = pltpu.make_async_copy(
                ov_ref, out_ref.at[pl.ds(rp * BR, BR), :], copy_sem
            )
            st.start()
            st.wait()

    grid = (R + 1, C)
    return pl.pallas_call(
        body,
        grid=grid,
        out_shape=jax.ShapeDtypeStruct((TOK, 2 * VS), jnp.float32),
        in_specs=[
            pl.BlockSpec(
                (BR, DM),
                lambda r, c: (jnp.minimum(r, R - 1), 0),
                memory_space=pltpu.VMEM,
            ),
            pl.BlockSpec(
                (DM, BC), lambda r, c: (0, c), memory_space=pltpu.VMEM
            ),
        ],
        out_specs=pl.BlockSpec(memory_space=pltpu.HBM),
        scratch_shapes=[
            pltpu.VMEM((2, BR, VS), jnp.bfloat16),
            pltpu.VMEM((NS, BR, VS), jnp.bfloat16),
            pltpu.VMEM((BR, 2 * VS), jnp.float32),
            pltpu.VMEM((2, BR, 1), jnp.float32),
            pltpu.SemaphoreType.DMA((2,)),
            pltpu.SemaphoreType.DMA((NS,)),
            pltpu.SemaphoreType.DMA,
        ],
        compiler_params=pltpu.CompilerParams(
            collective_id=0, vmem_limit_bytes=62 * 1024 * 1024
        ),
    )(x, W)


# device time: 607660 ns/iter; 1.4197x vs baseline; 1.4197x over previous
import jax
import jax.numpy as jnp
from jax import lax
from jax.experimental import pallas as pl
from jax.experimental.pallas import tpu as pltpu

TOK = 2048
DM = 4096
VS = 8192
BR = 128
BC = 1024
R = TOK // BR
C = VS // BC
CH = 2048
NS = 3


def kernel(x, W):
    W = W.astype(jnp.bfloat16)

    def body(
        x_ref,
        w_ref,
        out_ref,
        logits_ref,
        recv_ref,
        ov_ref,
        s_ref,
        xb_ref,
        send_sems,
        recv_sems,
        copy_sem,
    ):
        r = pl.program_id(0)
        c = pl.program_id(1)
        my_x = lax.axis_index("x")
        my_y = lax.axis_index("y")
        my_z = lax.axis_index("z")
        nbr = (1 - my_x, my_y, my_z)
        sslot = lax.rem(r, 2)

        def send_desc(s, rr):
            return pltpu.make_async_remote_copy(
                src_ref=logits_ref.at[s],
                dst_ref=recv_ref.at[lax.rem(rr, NS)],
                send_sem=send_sems.at[s],
                recv_sem=recv_sems.at[lax.rem(rr, NS)],
                device_id=nbr,
                device_id_type=pl.DeviceIdType.MESH,
            )

        @pl.when((r == 0) & (c == 0))
        def _():
            barrier = pltpu.get_barrier_semaphore()
            pl.semaphore_signal(
                barrier, inc=1, device_id=nbr,
                device_id_type=pl.DeviceIdType.MESH,
            )
            pl.semaphore_wait(barrier, 1)

        @pl.when((c == 0) & (r >= 2) & (r < R))
        def _():
            send_desc(sslot, r - 2).wait_send()

        @pl.when((c == 0) & (r == R))
        def _():
            send_desc(lax.rem(R - 2, 2), R - 2).wait_send()
            send_desc(lax.rem(R - 1, 2), R - 1).wait_send()

        @pl.when((r < R) & (c == 0))
        def _():
            xb_ref[...] = x_ref[...].astype(jnp.bfloat16)

        @pl.when(r < R)
        def _():
            et = jnp.exp(
                jnp.dot(
                    xb_ref[...], w_ref[...],
                    preferred_element_type=jnp.float32,
                )
            )
            logits_ref[sslot, :, pl.ds(c * BC, BC)] = et.astype(jnp.bfloat16)
            rs = jnp.sum(et, axis=1, keepdims=True)
            s_ref[sslot] = jnp.where(c == 0, rs, s_ref[sslot] + rs)

        @pl.when((c == C - 1) & (r < R))
        def _():
            send_desc(sslot, r).start()

        @pl.when((c == C - 1) & (r >= 1))
        def _():
            rp = r - 1
            pslot = lax.rem(rp, 2)
            rslot = lax.rem(rp, NS)
            send_desc(pslot, rp).wait_recv()

            off_loc = my_x * VS
            off_oth = (1 - my_x) * VS

            s = s_ref[pslot]
            for k in range(VS // CH):
                t = recv_ref[rslot, :, pl.ds(k * CH, CH)]
                s = s + jnp.sum(t.astype(jnp.float32), axis=1, keepdims=True)

            inv = 1.0 / s
            for k in range(VS // CH):
                t = logits_ref[pslot, :, pl.ds(k * CH, CH)]
                ov_ref[:, pl.ds(off_loc + k * CH, CH)] = (
                    t.astype(jnp.float32) * inv
                )
                t = recv_ref[rslot, :, pl.ds(k * CH, CH)]
                ov_ref[:, pl.ds(off_oth + k * CH, CH)] = (
                    t.astype(jnp.float32) * inv
                )

            st = pltpu.make_async_copy(
                ov_ref, out_ref.at[pl.ds(rp * BR, BR), :], copy_sem
            )
            st.start()
            st.wait()

    grid = (R + 1, C)
    return pl.pallas_call(
        body,
        grid=grid,
        out_shape=jax.ShapeDtypeStruct((TOK, 2 * VS), jnp.float32),
        in_specs=[
            pl.BlockSpec(
                (BR, DM),
                lambda r, c: (jnp.minimum(r, R - 1), 0),
                memory_space=pltpu.VMEM,
            ),
            pl.BlockSpec(
                (DM, BC), lambda r, c: (0, c), memory_space=pltpu.VMEM
            ),
        ],
        out_specs=pl.BlockSpec(memory_space=pltpu.HBM),
        scratch_shapes=[
            pltpu.VMEM((2, BR, VS), jnp.bfloat16),
            pltpu.VMEM((NS, BR, VS), jnp.bfloat16),
            pltpu.VMEM((BR, 2 * VS), jnp.float32),
            pltpu.VMEM((2, BR, 1), jnp.float32),
            pltpu.VMEM((BR, DM), jnp.bfloat16),
            pltpu.SemaphoreType.DMA((2,)),
            pltpu.SemaphoreType.DMA((NS,)),
            pltpu.SemaphoreType.DMA,
        ],
        compiler_params=pltpu.CompilerParams(
            collective_id=0, vmem_limit_bytes=62 * 1024 * 1024
        ),
    )(x, W)


# device time: 589648 ns/iter; 1.4630x vs baseline; 1.0305x over previous
import jax
import jax.numpy as jnp
from jax import lax
from jax.experimental import pallas as pl
from jax.experimental.pallas import tpu as pltpu

TOK = 2048
DM = 4096
VS = 8192
BR = 128
BC = 2048
R = TOK // BR
C = VS // BC
CH = 2048
NS = 3


def kernel(x, W):
    W = W.astype(jnp.bfloat16)

    def body(
        x_ref,
        w_ref,
        out_ref,
        logits_ref,
        recv_ref,
        ov_ref,
        s_ref,
        xb_ref,
        send_sems,
        recv_sems,
        copy_sem,
    ):
        r = pl.program_id(0)
        c = pl.program_id(1)
        my_x = lax.axis_index("x")
        my_y = lax.axis_index("y")
        my_z = lax.axis_index("z")
        nbr = (1 - my_x, my_y, my_z)
        sslot = lax.rem(r, 2)

        def send_desc(s, rr):
            return pltpu.make_async_remote_copy(
                src_ref=logits_ref.at[s],
                dst_ref=recv_ref.at[lax.rem(rr, NS)],
                send_sem=send_sems.at[s],
                recv_sem=recv_sems.at[lax.rem(rr, NS)],
                device_id=nbr,
                device_id_type=pl.DeviceIdType.MESH,
            )

        @pl.when((r == 0) & (c == 0))
        def _():
            barrier = pltpu.get_barrier_semaphore()
            pl.semaphore_signal(
                barrier, inc=1, device_id=nbr,
                device_id_type=pl.DeviceIdType.MESH,
            )
            pl.semaphore_wait(barrier, 1)

        @pl.when((c == 0) & (r >= 2) & (r < R))
        def _():
            send_desc(sslot, r - 2).wait_send()

        @pl.when((c == 0) & (r == R))
        def _():
            send_desc(lax.rem(R - 2, 2), R - 2).wait_send()
            send_desc(lax.rem(R - 1, 2), R - 1).wait_send()

        @pl.when((r < R) & (c == 0))
        def _():
            xb_ref[...] = x_ref[...].astype(jnp.bfloat16)

        @pl.when(r < R)
        def _():
            et = jnp.exp(
                jnp.dot(
                    xb_ref[...], w_ref[...],
                    preferred_element_type=jnp.float32,
                )
            )
            logits_ref[sslot, :, pl.ds(c * BC, BC)] = et.astype(jnp.bfloat16)
            rs = jnp.sum(et, axis=1, keepdims=True)
            s_ref[sslot] = jnp.where(c == 0, rs, s_ref[sslot] + rs)

        @pl.when((c == C - 1) & (r < R))
        def _():
            send_desc(sslot, r).start()

        @pl.when((c == C - 1) & (r >= 1))
        def _():
            rp = r - 1
            pslot = lax.rem(rp, 2)
            rslot = lax.rem(rp, NS)
            send_desc(pslot, rp).wait_recv()

            off_loc = my_x * VS
            off_oth = (1 - my_x) * VS

            s = s_ref[pslot]
            for k in range(VS // CH):
                t = recv_ref[rslot, :, pl.ds(k * CH, CH)]
                s = s + jnp.sum(t.astype(jnp.float32), axis=1, keepdims=True)

            inv = 1.0 / s
            for k in range(VS // CH):
                t = logits_ref[pslot, :, pl.ds(k * CH, CH)]
                ov_ref[:, pl.ds(off_loc + k * CH, CH)] = (
                    t.astype(jnp.float32) * inv
                )
                t = recv_ref[rslot, :, pl.ds(k * CH, CH)]
                ov_ref[:, pl.ds(off_oth + k * CH, CH)] = (
                    t.astype(jnp.float32) * inv
                )

            st = pltpu.make_async_copy(
                ov_ref, out_ref.at[pl.ds(rp * BR, BR), :], copy_sem
            )
            st.start()
            st.wait()

    grid = (R + 1, C)
    return pl.pallas_call(
        body,
        grid=grid,
        out_shape=jax.ShapeDtypeStruct((TOK, 2 * VS), jnp.float32),
        in_specs=[
            pl.BlockSpec(
                (BR, DM),
                lambda r, c: (jnp.minimum(r, R - 1), 0),
                memory_space=pltpu.VMEM,
            ),
            pl.BlockSpec(
                (DM, BC), lambda r, c: (0, c), memory_space=pltpu.VMEM
            ),
        ],
        out_specs=pl.BlockSpec(memory_space=pltpu.HBM),
        scratch_shapes=[
            pltpu.VMEM((2, BR, VS), jnp.bfloat16),
            pltpu.VMEM((NS, BR, VS), jnp.bfloat16),
            pltpu.VMEM((BR, 2 * VS), jnp.float32),
            pltpu.VMEM((2, BR, 1), jnp.float32),
            pltpu.VMEM((BR, DM), jnp.bfloat16),
            pltpu.SemaphoreType.DMA((2,)),
            pltpu.SemaphoreType.DMA((NS,)),
            pltpu.SemaphoreType.DMA,
        ],
        compiler_params=pltpu.CompilerParams(
            collective_id=0, vmem_limit_bytes=62 * 1024 * 1024
        ),
    )(x, W)
